# baseline (speedup 1.0000x reference)
import functools

import jax
import jax.numpy as jnp
from jax import lax
from jax.experimental import pallas as pl
from jax.experimental.pallas import tpu as pltpu
from jax.experimental.pallas import tpu_sc as plsc

N = 10000
E = 320000
F = 128
C = 128
ALPHA = 0.1
THETA = 0.5
EPS = 1e-5

NPAD = 10112
NUM_TILES = 32
CHUNK = 96
CHUNKS_PER_TILE = 106
EPADDED = NUM_TILES * CHUNKS_PER_TILE * CHUNK
ROWS_PER_SUB_PAD = NPAD // 16


def _seg_body(h_pad, e3s, e3d, zeros, out, src_idx, dst_idx, rows, acc,
              gsa, gsb, ssa, ssb):
    c = lax.axis_index("c")
    s = lax.axis_index("s")
    wid = c * 16 + s
    r0 = s * ROWS_PER_SUB_PAD
    pltpu.sync_copy(zeros.at[pl.ds(r0, ROWS_PER_SUB_PAD)],
                    acc.at[pl.ds(r0, ROWS_PER_SUB_PAD)])
    pltpu.sync_copy(e3s.at[wid], src_idx)
    pltpu.sync_copy(e3d.at[wid], dst_idx)
    plsc.subcore_barrier()

    def start_g(sem, j, b):
        off = pl.multiple_of(j * CHUNK, 8)
        pltpu.async_copy(h_pad.at[src_idx.at[pl.ds(off, CHUNK)]], rows.at[b], sem)

    def wait_g(sem, b):
        pltpu.make_async_copy(h_pad.at[pl.ds(0, CHUNK)], rows.at[b], sem).wait()

    def start_s(sem, j, b):
        pltpu.async_copy(rows.at[b], acc.at[dst_idx.at[j]], sem, add=True)

    def wait_s(sem, b):
        pltpu.make_async_copy(rows.at[b], acc.at[dst_idx.at[0]], sem).wait()

    start_g(gsa, 0, 0)
    start_g(gsb, 1, 1)

    def body(k, carry):
        wait_g(gsa, 0)
        start_s(ssa, 2 * k, 0)
        wait_g(gsb, 1)
        start_s(ssb, 2 * k + 1, 1)
        wait_s(ssa, 0)
        start_g(gsa, 2 * k + 2, 0)
        wait_s(ssb, 1)
        start_g(gsb, 2 * k + 3, 1)
        return carry

    lax.fori_loop(0, CHUNKS_PER_TILE // 2 - 1, body, 0)
    wait_g(gsa, 0)
    start_s(ssa, CHUNKS_PER_TILE - 2, 0)
    wait_g(gsb, 1)
    start_s(ssb, CHUNKS_PER_TILE - 1, 1)
    wait_s(ssa, 0)
    wait_s(ssb, 1)

    plsc.subcore_barrier()
    pltpu.sync_copy(acc.at[pl.ds(r0, ROWS_PER_SUB_PAD)],
                    out.at[c, pl.ds(r0, ROWS_PER_SUB_PAD)])


_seg_partials = pl.kernel(
    _seg_body,
    mesh=plsc.VectorSubcoreMesh(core_axis_name="c", subcore_axis_name="s"),
    out_type=jax.ShapeDtypeStruct((2, NPAD, C), jnp.float32),
    scratch_types=[
        pltpu.VMEM((CHUNKS_PER_TILE * CHUNK,), jnp.int32),
        pltpu.VMEM((CHUNKS_PER_TILE, CHUNK), jnp.int32),
        pltpu.VMEM((2, CHUNK, C), jnp.float32),
        pltpu.VMEM_SHARED((NPAD, C), jnp.float32),
        pltpu.SemaphoreType.DMA,
        pltpu.SemaphoreType.DMA,
        pltpu.SemaphoreType.DMA,
        pltpu.SemaphoreType.DMA,
    ],
)


def _k1_body(x_ref, w_ref, b_ref, o_ref):
    x0 = jnp.dot(x_ref[...], w_ref[...], preferred_element_type=jnp.float32)
    x0 = jnp.maximum(x0 + b_ref[...], 0.0)
    o_ref[0:N, :] = x0
    o_ref[N:NPAD, :] = jnp.zeros((NPAD - N, C), jnp.float32)


def _layer_body(beta_l, p_ref, x0_ref, w_ref, g_ref, be_ref, o_ref):
    agg = (p_ref[0] + p_ref[1]) * (1.0 - ALPHA) + ALPHA * x0_ref[...]
    h = agg * (1.0 - beta_l) + jnp.dot(
        agg, w_ref[...], preferred_element_type=jnp.float32) * beta_l
    row = lax.broadcasted_iota(jnp.int32, (NPAD, 1), 0)
    valid = row < N
    m = jnp.sum(h, axis=0, keepdims=True) / N
    d = jnp.where(valid, h - m, 0.0)
    v = jnp.sum(d * d, axis=0, keepdims=True) / N
    hn = d * lax.rsqrt(v + EPS) * g_ref[...] + be_ref[...]
    hn = jnp.maximum(hn, 0.0)
    o_ref[...] = jnp.where(valid, hn, 0.0)


def _head_body(h_ref, w1_ref, b1_ref, g_ref, be_ref, w2_ref, b2_ref, o_ref):
    h = h_ref[0:N, :]
    z = jnp.dot(h, w1_ref[...], preferred_element_type=jnp.float32) + b1_ref[...]
    m = jnp.sum(z, axis=0, keepdims=True) / N
    d = z - m
    v = jnp.sum(d * d, axis=0, keepdims=True) / N
    zn = d * lax.rsqrt(v + EPS) * g_ref[...] + be_ref[...]
    o_ref[...] = jnp.sum(zn * w2_ref[...], axis=1, keepdims=True) + b2_ref[...]


def kernel(x, edge_index, W_lin, b_lin, W_conv1, W_conv2, bn1_gamma, bn1_beta,
           W_lin1, b_lin1, bn2_gamma, bn2_beta, W_lin2, b_lin2):
    import numpy as np
    pad = jnp.full((2, EPADDED - E), N, dtype=jnp.int32)
    epad = jnp.concatenate([edge_index, pad], axis=1)
    e3s = epad[0].reshape(NUM_TILES, CHUNKS_PER_TILE * CHUNK)
    e3d = epad[1].reshape(NUM_TILES, CHUNKS_PER_TILE, CHUNK)
    zeros = jnp.zeros((NPAD, C), jnp.float32)

    x0p = pl.pallas_call(
        _k1_body,
        out_shape=jax.ShapeDtypeStruct((NPAD, C), jnp.float32),
    )(x, W_lin, b_lin.reshape(1, C))

    h = x0p
    for layer, W in enumerate([W_conv1, W_conv2], start=1):
        beta_l = float(np.log(THETA / layer + 1.0))
        parts = _seg_partials(h, e3s, e3d, zeros)
        h = pl.pallas_call(
            functools.partial(_layer_body, beta_l),
            out_shape=jax.ShapeDtypeStruct((NPAD, C), jnp.float32),
        )(parts, x0p, W, bn1_gamma.reshape(1, C), bn1_beta.reshape(1, C))

    out = pl.pallas_call(
        _head_body,
        out_shape=jax.ShapeDtypeStruct((N, 1), jnp.float32),
    )(h, W_lin1, b_lin1.reshape(1, 16), bn2_gamma.reshape(1, 16),
      bn2_beta.reshape(1, 16), W_lin2.reshape(1, 16), b_lin2.reshape(1, 1))
    return out


# speedup vs baseline: 1.7616x; 1.7616x over previous
import functools

import jax
import jax.numpy as jnp
from jax import lax
from jax.experimental import pallas as pl
from jax.experimental.pallas import tpu as pltpu
from jax.experimental.pallas import tpu_sc as plsc

N = 10000
E = 320000
F = 128
C = 128
HALF = C // 2
ALPHA = 0.1
THETA = 0.5
EPS = 1e-5

NPAD = 10112
SCHUNK = 72
SCHUNKS_PER_TILE = 140
EPADDED = 32 * SCHUNKS_PER_TILE * SCHUNK
ROWS_PER_SUB = NPAD // 16


def _seg_body(h_pad, esrc, edst, zeros, out, src_idx, dstage, rows, acc,
              gsem, ssem, isem):
    c = lax.axis_index("c")
    s = lax.axis_index("s")
    wid = c * 16 + s
    r0 = s * ROWS_PER_SUB
    d0 = wid * SCHUNKS_PER_TILE
    pltpu.sync_copy(zeros.at[pl.ds(r0, ROWS_PER_SUB)],
                    acc.at[pl.ds(r0, ROWS_PER_SUB)])
    pltpu.sync_copy(esrc.at[wid], src_idx)
    plsc.subcore_barrier()

    def start_stage(j, b):
        pltpu.async_copy(edst.at[d0 + j], dstage.at[b], isem)

    def wait_stage(b):
        pltpu.make_async_copy(edst.at[0], dstage.at[b], isem).wait()

    def start_gather(j, b):
        off = pl.multiple_of(j * SCHUNK, 8)
        pltpu.async_copy(h_pad.at[src_idx.at[pl.ds(off, 40)]],
                         rows.at[b, pl.ds(0, 40)], gsem)
        pltpu.async_copy(h_pad.at[src_idx.at[pl.ds(off + 40, 32)]],
                         rows.at[b, pl.ds(40, 32)], gsem)

    def wait_gather(b):
        pltpu.make_async_copy(h_pad.at[pl.ds(0, 40)],
                              rows.at[b, pl.ds(0, 40)], gsem).wait()
        pltpu.make_async_copy(h_pad.at[pl.ds(0, 32)],
                              rows.at[b, pl.ds(40, 32)], gsem).wait()

    def start_scatter(b):
        pltpu.async_copy(rows.at[b], acc.at[dstage.at[b]], ssem, add=True)

    def wait_scatter(b):
        pltpu.make_async_copy(rows.at[b], acc.at[dstage.at[0]], ssem).wait()

    def sub(j, b, prefetch):
        wait_gather(b)
        wait_stage(b)
        start_scatter(b)
        if prefetch:
            b2 = (b + 2) % 4
            wait_scatter(b2)
            start_stage(j + 2, b2)
            start_gather(j + 2, b2)

    start_stage(0, 0)
    start_gather(0, 0)
    start_stage(1, 1)
    start_gather(1, 1)
    wait_gather(0); wait_stage(0); start_scatter(0)
    start_stage(2, 2); start_gather(2, 2)
    wait_gather(1); wait_stage(1); start_scatter(1)
    start_stage(3, 3); start_gather(3, 3)

    def body(g, carry):
        j = 4 * g + 2
        sub(j, 2, True)
        sub(j + 1, 3, True)
        sub(j + 2, 0, True)
        sub(j + 3, 1, True)
        return carry

    lax.fori_loop(0, (SCHUNKS_PER_TILE - 4) // 4, body, 0)
    sub(SCHUNKS_PER_TILE - 2, 2, False)
    sub(SCHUNKS_PER_TILE - 1, 3, False)
    wait_scatter(0)
    wait_scatter(1)
    wait_scatter(2)
    wait_scatter(3)

    plsc.subcore_barrier()
    pltpu.sync_copy(acc.at[pl.ds(r0, ROWS_PER_SUB)],
                    out.at[c, pl.ds(r0, ROWS_PER_SUB)])


_seg_partials = pl.kernel(
    _seg_body,
    mesh=plsc.VectorSubcoreMesh(core_axis_name="c", subcore_axis_name="s"),
    out_type=jax.ShapeDtypeStruct((2, NPAD, C), jnp.float32),
    scratch_types=[
        pltpu.VMEM((SCHUNKS_PER_TILE * SCHUNK,), jnp.int32),
        pltpu.VMEM((4, SCHUNK), jnp.int32),
        pltpu.VMEM((4, SCHUNK, C), jnp.float32),
        pltpu.VMEM_SHARED((NPAD, C), jnp.float32),
        pltpu.SemaphoreType.DMA,
        pltpu.SemaphoreType.DMA,
        pltpu.SemaphoreType.DMA,
    ],
)


def _k1_body(x_ref, w_ref, b_ref, x0_ref):
    x0 = jnp.dot(x_ref[...], w_ref[...], preferred_element_type=jnp.float32)
    x0 = jnp.maximum(x0 + b_ref[...], 0.0)
    x0_ref[0:N, :] = x0
    x0_ref[N:NPAD, :] = jnp.zeros((NPAD - N, C), jnp.float32)


def _layer_body(beta_l, p_ref, x0_ref, w_ref, g_ref, be_ref, h_ref):
    seg = p_ref[0] + p_ref[1]
    agg = seg * (1.0 - ALPHA) + ALPHA * x0_ref[...]
    h = agg * (1.0 - beta_l) + jnp.dot(
        agg, w_ref[...], preferred_element_type=jnp.float32) * beta_l
    row = lax.broadcasted_iota(jnp.int32, (NPAD, 1), 0)
    valid = row < N
    m = jnp.sum(h, axis=0, keepdims=True) / N
    d = jnp.where(valid, h - m, 0.0)
    v = jnp.sum(d * d, axis=0, keepdims=True) / N
    hn = d * lax.rsqrt(v + EPS) * g_ref[...] + be_ref[...]
    hn = jnp.maximum(hn, 0.0)
    h_ref[...] = jnp.where(valid, hn, 0.0)


def _head_body(h_ref, w1_ref, b1_ref, g_ref, be_ref, w2_ref, b2_ref, o_ref):
    h = h_ref[0:N, :]
    z = jnp.dot(h, w1_ref[...], preferred_element_type=jnp.float32) + b1_ref[...]
    m = jnp.sum(z, axis=0, keepdims=True) / N
    d = z - m
    v = jnp.sum(d * d, axis=0, keepdims=True) / N
    zn = d * lax.rsqrt(v + EPS) * g_ref[...] + be_ref[...]
    o_ref[...] = jnp.sum(zn * w2_ref[...], axis=1, keepdims=True) + b2_ref[...]


def kernel(x, edge_index, W_lin, b_lin, W_conv1, W_conv2, bn1_gamma, bn1_beta,
           W_lin1, b_lin1, bn2_gamma, bn2_beta, W_lin2, b_lin2):
    import numpy as np
    pad = jnp.full((2, EPADDED - E), N, dtype=jnp.int32)
    epad = jnp.concatenate([edge_index, pad], axis=1)
    esrc = epad[0].reshape(32, SCHUNKS_PER_TILE * SCHUNK)
    edst = epad[1].reshape(32 * SCHUNKS_PER_TILE, SCHUNK)
    zeros = jnp.zeros((NPAD, C), jnp.float32)

    x0p = pl.pallas_call(
        _k1_body,
        out_shape=jax.ShapeDtypeStruct((NPAD, C), jnp.float32),
    )(x, W_lin, b_lin.reshape(1, C))

    h = x0p
    for layer, W in enumerate([W_conv1, W_conv2], start=1):
        beta_l = float(np.log(THETA / layer + 1.0))
        parts = _seg_partials(h, esrc, edst, zeros)
        h = pl.pallas_call(
            functools.partial(_layer_body, beta_l),
            out_shape=jax.ShapeDtypeStruct((NPAD, C), jnp.float32),
        )(parts, x0p, W, bn1_gamma.reshape(1, C), bn1_beta.reshape(1, C))

    out = pl.pallas_call(
        _head_body,
        out_shape=jax.ShapeDtypeStruct((N, 1), jnp.float32),
    )(h, W_lin1, b_lin1.reshape(1, 16), bn2_gamma.reshape(1, 16),
      bn2_beta.reshape(1, 16), W_lin2.reshape(1, 16), b_lin2.reshape(1, 1))
    return out
